# baseline (speedup 1.0000x reference)
import functools

import jax
import jax.numpy as jnp
from jax import lax
from jax.experimental import pallas as pl
from jax.experimental.pallas import tpu as pltpu
from jax.experimental.pallas import tpu_sc as plsc

VOCAB_N = 100000
EMB_N = 300
B_N = 4096
L_N = 200

NC = 2
NS = 16
LANES = 16
NW = NC * NS

SCP_PER = 2176
V_SC = NW * SCP_PER
VT = VOCAB_N - V_SC
SCP_GROUPS = SCP_PER // 16
SCP_PAIRS = SCP_GROUPS // 2

CHUNK = 4096
NCHUNK = (VT + CHUNK - 1) // CHUNK
NBUF = 4
_CHUNK_ROWS = [CHUNK] * (NCHUNK - 1) + [VT - CHUNK * (NCHUNK - 1)]
VT_PAD = ((VT + 127) // 128) * 128


def _mm_body(w_ref, t_hbm, o_ref, buf, sems):
    def start(i):
        rows = _CHUNK_ROWS[i]
        pltpu.make_async_copy(
            t_hbm.at[pl.ds(i * CHUNK, rows), :],
            buf.at[i % NBUF, pl.ds(0, rows), :],
            sems.at[i % NBUF]).start()

    for i in range(NBUF):
        start(i)
    for i in range(NCHUNK):
        rows = _CHUNK_ROWS[i]
        pltpu.make_async_copy(
            t_hbm.at[pl.ds(i * CHUNK, rows), :],
            buf.at[i % NBUF, pl.ds(0, rows), :],
            sems.at[i % NBUF]).wait()
        o_ref[:, pl.ds(i * CHUNK, rows)] = lax.dot_general(
            w_ref[...], buf[i % NBUF, pl.ds(0, rows), :],
            dimension_numbers=(((1,), (1,)), ((), ())),
            preferred_element_type=jnp.float32)
        if i + NBUF < NCHUNK:
            start(i + NBUF)
    o_ref[:, 0:1] = jnp.zeros((8, 1), jnp.float32)


def _project_table_tc(Wp, table):
    return pl.pallas_call(
        _mm_body,
        in_specs=[
            pl.BlockSpec(memory_space=pltpu.VMEM),
            pl.BlockSpec(memory_space=pl.ANY),
        ],
        out_specs=pl.BlockSpec(memory_space=pltpu.VMEM),
        out_shape=jax.ShapeDtypeStruct((8, CHUNK * NCHUNK), jnp.float32),
        scratch_shapes=[
            pltpu.VMEM((NBUF, CHUNK, EMB_N), jnp.float32),
            pltpu.SemaphoreType.DMA((NBUF,)),
        ],
    )(Wp, table)


def _make_sc_project():
    mesh = plsc.VectorSubcoreMesh(core_axis_name="c", subcore_axis_name="s")

    @functools.partial(
        pl.kernel,
        mesh=mesh,
        compiler_params=pltpu.CompilerParams(needs_layout_passes=False),
        out_type=jax.ShapeDtypeStruct((2 * V_SC,), jnp.float32),
        scratch_types=[
            pltpu.VMEM((LANES, EMB_N), jnp.float32),
            pltpu.VMEM((LANES, EMB_N), jnp.float32),
            pltpu.VMEM((2 * EMB_N + 8,), jnp.float32),
            pltpu.VMEM((SCP_PER,), jnp.float32),
            pltpu.VMEM((SCP_PER,), jnp.float32),
            pltpu.SemaphoreType.DMA,
            pltpu.SemaphoreType.DMA,
        ],
    )
    def sc_project(t_hbm, wf_hbm, out_hbm, bufa, bufb, wf_v, c0_v, c1_v,
                   sema, semb):
        wid = lax.axis_index("s") * NC + lax.axis_index("c")
        base = VT + wid * SCP_PER
        pltpu.sync_copy(wf_hbm, wf_v)
        rows16 = lax.iota(jnp.int32, 16)

        def start(g, buf, sem):
            pltpu.make_async_copy(
                t_hbm.at[pl.ds(base + g * LANES, LANES), :], buf, sem).start()

        def wait(g, buf, sem):
            pltpu.make_async_copy(
                t_hbm.at[pl.ds(base + g * LANES, LANES), :], buf, sem).wait()

        def process(g, buf):
            acc0 = jnp.zeros((LANES,), jnp.float32)
            acc1 = jnp.zeros((LANES,), jnp.float32)
            for eb in range(0, EMB_N, LANES):
                n = min(LANES, EMB_N - eb)
                w0blk = wf_v[pl.ds(eb, 16)]
                w1blk = wf_v[pl.ds(EMB_N + eb, 16)]
                for l in range(n):
                    col = jnp.full((LANES,), eb + l, jnp.int32)
                    v = plsc.load_gather(buf, [rows16, col])
                    acc0 = acc0 + v * w0blk[l]
                    acc1 = acc1 + v * w1blk[l]
            c0_v[pl.ds(g * LANES, LANES)] = acc0
            c1_v[pl.ds(g * LANES, LANES)] = acc1

        def grp1(g, carry):
            start(g, bufa, sema)
            wait(g, bufa, sema)
            process(g, bufa)
            return carry

        lax.fori_loop(0, SCP_GROUPS, grp1, 0)
        pltpu.sync_copy(c0_v, out_hbm.at[pl.ds(wid * SCP_PER, SCP_PER)])
        pltpu.sync_copy(c1_v, out_hbm.at[pl.ds(V_SC + wid * SCP_PER,
                                               SCP_PER)])

    return sc_project


_sc_project = _make_sc_project()


ROWS_PER_WORKER = B_N // NS
GROUPS_PER_WORKER = ROWS_PER_WORKER // LANES
GROUP_WORDS = LANES * L_N


def _make_sc_gather():
    mesh = plsc.VectorSubcoreMesh(core_axis_name="c", subcore_axis_name="s")

    @functools.partial(
        pl.kernel,
        mesh=mesh,
        compiler_params=pltpu.CompilerParams(needs_layout_passes=False),
        out_type=jax.ShapeDtypeStruct((2, B_N), jnp.float32),
        scratch_types=[
            pltpu.VMEM((VOCAB_N,), jnp.float32),
            pltpu.VMEM((GROUP_WORDS,), jnp.int32),
            pltpu.VMEM((ROWS_PER_WORKER,), jnp.float32),
            pltpu.VMEM((LANES,), jnp.float32),
            pltpu.SemaphoreType.DMA,
            pltpu.SemaphoreType.DMA,
        ],
    )
    def sc_gather(tc_hbm, sc_hbm, x_hbm, bb_hbm, out_hbm,
                  col_v, idx_v, out_v, b_v, sem_tc, sem_sc):
        cls = lax.axis_index("c")
        w2 = lax.axis_index("s")
        tc_cp = pltpu.make_async_copy(
            tc_hbm.at[pl.ds(cls * VT, VT)], col_v.at[pl.ds(0, VT)],
            sem_tc)
        tc_cp.start()
        sc_cp = pltpu.make_async_copy(
            sc_hbm.at[pl.ds(cls * V_SC, V_SC)], col_v.at[pl.ds(VT, V_SC)],
            sem_sc)
        sc_cp.start()
        pltpu.sync_copy(bb_hbm.at[cls], b_v)
        tc_cp.wait()
        sc_cp.wait()
        bvec = b_v[...]
        rowoff = lax.iota(jnp.int32, 16) * L_N

        def grp(g, carry):
            gbase = (w2 * GROUPS_PER_WORKER + g) * GROUP_WORDS
            pltpu.sync_copy(x_hbm.at[pl.ds(gbase, GROUP_WORDS)], idx_v)
            acc = jnp.zeros((LANES,), jnp.float32)
            for j in range(L_N):
                idxs = plsc.load_gather(idx_v, [rowoff + j])
                acc = acc + plsc.load_gather(col_v, [idxs])
            z = acc * jnp.float32(1.0 / L_N) + bvec
            out_v[pl.ds(g * LANES, LANES)] = (
                jnp.float32(1.0) / (jnp.float32(1.0) + jnp.exp(-z)))
            return carry

        lax.fori_loop(0, GROUPS_PER_WORKER, grp, 0)
        pltpu.sync_copy(out_v, out_hbm.at[cls, pl.ds(w2 * ROWS_PER_WORKER,
                                                     ROWS_PER_WORKER)])

    return sc_gather


_sc_gather = _make_sc_gather()


def kernel(x, table, W, b):
    xi = x.astype(jnp.int32).reshape(-1)
    tf = table.astype(jnp.float32)
    Wf = W.astype(jnp.float32)
    Wp = jnp.pad(Wf, ((0, 8 - Wf.shape[0]), (0, 0)))
    small_tc = _project_table_tc(Wp, tf)[:2, :VT].reshape(-1)
    small_sc = _sc_project(
        tf, jnp.concatenate([Wf.reshape(-1), jnp.zeros((8,), jnp.float32)]))
    bb = jnp.broadcast_to(b.astype(jnp.float32)[:, None], (2, LANES))
    out2 = _sc_gather(small_tc, small_sc, xi, bb)
    return out2.T


# speedup vs baseline: 1.2872x; 1.2872x over previous
import functools

import jax
import jax.numpy as jnp
from jax import lax
from jax.experimental import pallas as pl
from jax.experimental.pallas import tpu as pltpu
from jax.experimental.pallas import tpu_sc as plsc

VOCAB_N = 100000
EMB_N = 300
B_N = 4096
L_N = 200

NC = 2
NS = 16
LANES = 16
NW = NC * NS

SCP_PER = 2176
V_SC = NW * SCP_PER
VT = VOCAB_N - V_SC
SCP_GROUPS = SCP_PER // 16
SCP_PAIRS = SCP_GROUPS // 2

CHUNK = 4096
NCHUNK = (VT + CHUNK - 1) // CHUNK
NBUF = 4
_CHUNK_ROWS = [CHUNK] * (NCHUNK - 1) + [VT - CHUNK * (NCHUNK - 1)]
VT_PAD = ((VT + 127) // 128) * 128


def _mm_body(w_ref, t_hbm, o_ref, buf, sems):
    def start(i):
        rows = _CHUNK_ROWS[i]
        pltpu.make_async_copy(
            t_hbm.at[pl.ds(i * CHUNK, rows), :],
            buf.at[i % NBUF, pl.ds(0, rows), :],
            sems.at[i % NBUF]).start()

    for i in range(NBUF):
        start(i)
    for i in range(NCHUNK):
        rows = _CHUNK_ROWS[i]
        pltpu.make_async_copy(
            t_hbm.at[pl.ds(i * CHUNK, rows), :],
            buf.at[i % NBUF, pl.ds(0, rows), :],
            sems.at[i % NBUF]).wait()
        o_ref[:, pl.ds(i * CHUNK, rows)] = lax.dot_general(
            w_ref[...], buf[i % NBUF, pl.ds(0, rows), :],
            dimension_numbers=(((1,), (1,)), ((), ())),
            preferred_element_type=jnp.float32)
        if i + NBUF < NCHUNK:
            start(i + NBUF)
    o_ref[:, 0:1] = jnp.zeros((8, 1), jnp.float32)


def _project_table_tc(Wp, table):
    return pl.pallas_call(
        _mm_body,
        in_specs=[
            pl.BlockSpec(memory_space=pltpu.VMEM),
            pl.BlockSpec(memory_space=pl.ANY),
        ],
        out_specs=pl.BlockSpec(memory_space=pltpu.VMEM),
        out_shape=jax.ShapeDtypeStruct((8, CHUNK * NCHUNK), jnp.float32),
        scratch_shapes=[
            pltpu.VMEM((NBUF, CHUNK, EMB_N), jnp.float32),
            pltpu.SemaphoreType.DMA((NBUF,)),
        ],
    )(Wp, table)


def _make_sc_project():
    mesh = plsc.VectorSubcoreMesh(core_axis_name="c", subcore_axis_name="s")

    @functools.partial(
        pl.kernel,
        mesh=mesh,
        compiler_params=pltpu.CompilerParams(needs_layout_passes=False),
        out_type=jax.ShapeDtypeStruct((2 * V_SC,), jnp.float32),
        scratch_types=[
            pltpu.VMEM((LANES, EMB_N), jnp.float32),
            pltpu.VMEM((LANES, EMB_N), jnp.float32),
            pltpu.VMEM((2 * EMB_N, LANES), jnp.float32),
            pltpu.VMEM((SCP_PER,), jnp.float32),
            pltpu.VMEM((SCP_PER,), jnp.float32),
            pltpu.SemaphoreType.DMA,
            pltpu.SemaphoreType.DMA,
        ],
    )
    def sc_project(t_hbm, ws_hbm, out_hbm, bufa, bufb, ws_v, c0_v, c1_v,
                   sema, semb):
        wid = lax.axis_index("s") * NC + lax.axis_index("c")
        base = VT + wid * SCP_PER
        pltpu.sync_copy(ws_hbm, ws_v)
        rows16 = lax.iota(jnp.int32, 16)

        def start(g, buf, sem):
            pltpu.make_async_copy(
                t_hbm.at[pl.ds(base + g * LANES, LANES), :], buf, sem).start()

        def wait(g, buf, sem):
            pltpu.make_async_copy(
                t_hbm.at[pl.ds(base + g * LANES, LANES), :], buf, sem).wait()

        EUNROLL = 20

        def process(g, buf):
            def echunk(m, accs):
                a0, a1 = accs
                base_e = m * EUNROLL
                for l in range(EUNROLL):
                    col = jnp.full((LANES,), l, jnp.int32) + base_e
                    v = plsc.load_gather(buf, [rows16, col])
                    a0 = a0 + v * ws_v[base_e + l]
                    a1 = a1 + v * ws_v[EMB_N + base_e + l]
                return (a0, a1)

            z16 = jnp.zeros((LANES,), jnp.float32)
            acc0, acc1 = lax.fori_loop(0, EMB_N // EUNROLL, echunk,
                                       (z16, z16))
            c0_v[pl.ds(g * LANES, LANES)] = acc0
            c1_v[pl.ds(g * LANES, LANES)] = acc1

        start(0, bufa, sema)

        def pair(p, carry):
            g = p * 2
            start(g + 1, bufb, semb)
            wait(g, bufa, sema)
            process(g, bufa)

            @pl.when(p + 1 < SCP_PAIRS)
            def _():
                start(g + 2, bufa, sema)

            wait(g + 1, bufb, semb)
            process(g + 1, bufb)
            return carry

        lax.fori_loop(0, SCP_PAIRS, pair, 0)
        pltpu.sync_copy(c0_v, out_hbm.at[pl.ds(wid * SCP_PER, SCP_PER)])
        pltpu.sync_copy(c1_v, out_hbm.at[pl.ds(V_SC + wid * SCP_PER,
                                               SCP_PER)])

    return sc_project


_sc_project = _make_sc_project()


ROWS_PER_WORKER = B_N // NS
GROUPS_PER_WORKER = ROWS_PER_WORKER // LANES
GROUP_WORDS = LANES * L_N


def _make_sc_gather():
    mesh = plsc.VectorSubcoreMesh(core_axis_name="c", subcore_axis_name="s")

    @functools.partial(
        pl.kernel,
        mesh=mesh,
        compiler_params=pltpu.CompilerParams(needs_layout_passes=False),
        out_type=jax.ShapeDtypeStruct((2, B_N), jnp.float32),
        scratch_types=[
            pltpu.VMEM((VOCAB_N,), jnp.float32),
            pltpu.VMEM((GROUP_WORDS,), jnp.int32),
            pltpu.VMEM((ROWS_PER_WORKER,), jnp.float32),
            pltpu.VMEM((LANES,), jnp.float32),
            pltpu.SemaphoreType.DMA,
            pltpu.SemaphoreType.DMA,
        ],
    )
    def sc_gather(tc_hbm, sc_hbm, x_hbm, bb_hbm, out_hbm,
                  col_v, idx_v, out_v, b_v, sem_tc, sem_sc):
        cls = lax.axis_index("c")
        w2 = lax.axis_index("s")
        tc_cp = pltpu.make_async_copy(
            tc_hbm.at[pl.ds(cls * VT, VT)], col_v.at[pl.ds(0, VT)],
            sem_tc)
        tc_cp.start()
        sc_cp = pltpu.make_async_copy(
            sc_hbm.at[pl.ds(cls * V_SC, V_SC)], col_v.at[pl.ds(VT, V_SC)],
            sem_sc)
        sc_cp.start()
        pltpu.sync_copy(bb_hbm.at[cls], b_v)
        tc_cp.wait()
        sc_cp.wait()
        bvec = b_v[...]
        rowoff = lax.iota(jnp.int32, 16) * L_N

        def grp(g, carry):
            gbase = (w2 * GROUPS_PER_WORKER + g) * GROUP_WORDS
            pltpu.sync_copy(x_hbm.at[pl.ds(gbase, GROUP_WORDS)], idx_v)
            acc = jnp.zeros((LANES,), jnp.float32)
            for j in range(L_N):
                idxs = plsc.load_gather(idx_v, [rowoff + j])
                acc = acc + plsc.load_gather(col_v, [idxs])
            z = acc * jnp.float32(1.0 / L_N) + bvec
            out_v[pl.ds(g * LANES, LANES)] = (
                jnp.float32(1.0) / (jnp.float32(1.0) + jnp.exp(-z)))
            return carry

        lax.fori_loop(0, GROUPS_PER_WORKER, grp, 0)
        pltpu.sync_copy(out_v, out_hbm.at[cls, pl.ds(w2 * ROWS_PER_WORKER,
                                                     ROWS_PER_WORKER)])

    return sc_gather


_sc_gather = _make_sc_gather()


def kernel(x, table, W, b):
    xi = x.astype(jnp.int32).reshape(-1)
    tf = table.astype(jnp.float32)
    Wf = W.astype(jnp.float32)
    Wp = jnp.pad(Wf, ((0, 8 - Wf.shape[0]), (0, 0)))
    small_tc = _project_table_tc(Wp, tf)[:2, :VT].reshape(-1)
    wsplat = jnp.broadcast_to(Wf.reshape(-1)[:, None], (2 * EMB_N, LANES))
    small_sc = _sc_project(tf, wsplat)
    bb = jnp.broadcast_to(b.astype(jnp.float32)[:, None], (2, LANES))
    out2 = _sc_gather(small_tc, small_sc, xi, bb)
    return out2.T


# speedup vs baseline: 2.2778x; 1.7696x over previous
import functools

import jax
import jax.numpy as jnp
from jax import lax
from jax.experimental import pallas as pl
from jax.experimental.pallas import tpu as pltpu
from jax.experimental.pallas import tpu_sc as plsc

VOCAB_N = 100000
EMB_N = 300
B_N = 4096
L_N = 200

NC = 2
NS = 16
LANES = 16
NW = NC * NS

SCP_PER = 2176
V_SC = NW * SCP_PER
VT = VOCAB_N - V_SC
SCP_GROUPS = SCP_PER // 16
SCP_PAIRS = SCP_GROUPS // 2

CHUNK = 4096
NCHUNK = (VT + CHUNK - 1) // CHUNK
NBUF = 4
_CHUNK_ROWS = [CHUNK] * (NCHUNK - 1) + [VT - CHUNK * (NCHUNK - 1)]
VT_PAD = ((VT + 127) // 128) * 128


def _mm_body(w_ref, t_hbm, o_ref, buf, sems):
    def start(i):
        rows = _CHUNK_ROWS[i]
        pltpu.make_async_copy(
            t_hbm.at[pl.ds(i * CHUNK, rows), :],
            buf.at[i % NBUF, pl.ds(0, rows), :],
            sems.at[i % NBUF]).start()

    for i in range(NBUF):
        start(i)
    for i in range(NCHUNK):
        rows = _CHUNK_ROWS[i]
        pltpu.make_async_copy(
            t_hbm.at[pl.ds(i * CHUNK, rows), :],
            buf.at[i % NBUF, pl.ds(0, rows), :],
            sems.at[i % NBUF]).wait()
        o_ref[:, pl.ds(i * CHUNK, rows)] = lax.dot_general(
            w_ref[...], buf[i % NBUF, pl.ds(0, rows), :],
            dimension_numbers=(((1,), (1,)), ((), ())),
            preferred_element_type=jnp.float32)
        if i + NBUF < NCHUNK:
            start(i + NBUF)
    o_ref[:, 0:1] = jnp.zeros((8, 1), jnp.float32)


def _project_table_tc(Wp, table):
    return pl.pallas_call(
        _mm_body,
        in_specs=[
            pl.BlockSpec(memory_space=pltpu.VMEM),
            pl.BlockSpec(memory_space=pl.ANY),
        ],
        out_specs=pl.BlockSpec(memory_space=pltpu.VMEM),
        out_shape=jax.ShapeDtypeStruct((8, CHUNK * NCHUNK), jnp.float32),
        scratch_shapes=[
            pltpu.VMEM((NBUF, CHUNK, EMB_N), jnp.float32),
            pltpu.SemaphoreType.DMA((NBUF,)),
        ],
    )(Wp, table)


def _make_sc_project():
    mesh = plsc.VectorSubcoreMesh(core_axis_name="c", subcore_axis_name="s")

    @functools.partial(
        pl.kernel,
        mesh=mesh,
        compiler_params=pltpu.CompilerParams(needs_layout_passes=False),
        out_type=jax.ShapeDtypeStruct((2 * V_SC,), jnp.float32),
        scratch_types=[
            pltpu.VMEM((LANES, EMB_N), jnp.float32),
            pltpu.VMEM((LANES, EMB_N), jnp.float32),
            pltpu.VMEM((2 * 304,), jnp.float32),
            pltpu.VMEM((LANES, LANES), jnp.float32),
            pltpu.VMEM((LANES, LANES), jnp.float32),
            pltpu.VMEM((SCP_PER,), jnp.float32),
            pltpu.VMEM((SCP_PER,), jnp.float32),
            pltpu.SemaphoreType.DMA,
            pltpu.SemaphoreType.DMA,
        ],
    )
    def sc_project(t_hbm, wg_hbm, out_hbm, bufa, bufb, wg_v, m0_v, m1_v,
                   c0_v, c1_v, sema, semb):
        wid = lax.axis_index("s") * NC + lax.axis_index("c")
        base = VT + wid * SCP_PER
        pltpu.sync_copy(wg_hbm, wg_v)
        rows16 = lax.iota(jnp.int32, 16)
        NREG = EMB_N // LANES
        w0 = [wg_v[pl.ds(16 * k, 16)] for k in range(NREG + 1)]
        w1 = [wg_v[pl.ds(304 + 16 * k, 16)] for k in range(NREG + 1)]

        def start(g, buf, sem):
            pltpu.make_async_copy(
                t_hbm.at[pl.ds(base + g * LANES, LANES), :], buf, sem).start()

        def wait(g, buf, sem):
            pltpu.make_async_copy(
                t_hbm.at[pl.ds(base + g * LANES, LANES), :], buf, sem).wait()

        def process(g, buf):
            for r in range(LANES):
                v0 = buf[r, pl.ds(0, 16)]
                acc0 = v0 * w0[0]
                acc1 = v0 * w1[0]
                for k in range(1, NREG):
                    v = buf[r, pl.ds(16 * k, 16)]
                    acc0 = acc0 + v * w0[k]
                    acc1 = acc1 + v * w1[k]
                vt = buf[r, pl.ds(EMB_N - 16, 16)]
                acc0 = acc0 + vt * w0[NREG]
                acc1 = acc1 + vt * w1[NREG]
                m0_v[r, pl.ds(0, 16)] = acc0
                m1_v[r, pl.ds(0, 16)] = acc1
            col0 = jnp.full((LANES,), 0, jnp.int32)
            tot0 = plsc.load_gather(m0_v, [rows16, col0])
            tot1 = plsc.load_gather(m1_v, [rows16, col0])
            for c in range(1, LANES):
                colc = jnp.full((LANES,), c, jnp.int32)
                tot0 = tot0 + plsc.load_gather(m0_v, [rows16, colc])
                tot1 = tot1 + plsc.load_gather(m1_v, [rows16, colc])
            c0_v[pl.ds(g * LANES, LANES)] = tot0
            c1_v[pl.ds(g * LANES, LANES)] = tot1

        start(0, bufa, sema)

        def pair(p, carry):
            g = p * 2
            start(g + 1, bufb, semb)
            wait(g, bufa, sema)
            process(g, bufa)

            @pl.when(p + 1 < SCP_PAIRS)
            def _():
                start(g + 2, bufa, sema)

            wait(g + 1, bufb, semb)
            process(g + 1, bufb)
            return carry

        lax.fori_loop(0, SCP_PAIRS, pair, 0)
        pltpu.sync_copy(c0_v, out_hbm.at[pl.ds(wid * SCP_PER, SCP_PER)])
        pltpu.sync_copy(c1_v, out_hbm.at[pl.ds(V_SC + wid * SCP_PER,
                                               SCP_PER)])

    return sc_project


_sc_project = _make_sc_project()


ROWS_PER_WORKER = B_N // NS
GROUPS_PER_WORKER = ROWS_PER_WORKER // LANES
GROUP_WORDS = LANES * L_N


def _make_sc_gather():
    mesh = plsc.VectorSubcoreMesh(core_axis_name="c", subcore_axis_name="s")

    @functools.partial(
        pl.kernel,
        mesh=mesh,
        compiler_params=pltpu.CompilerParams(needs_layout_passes=False),
        out_type=jax.ShapeDtypeStruct((2, B_N), jnp.float32),
        scratch_types=[
            pltpu.VMEM((VOCAB_N,), jnp.float32),
            pltpu.VMEM((GROUP_WORDS,), jnp.int32),
            pltpu.VMEM((ROWS_PER_WORKER,), jnp.float32),
            pltpu.VMEM((LANES,), jnp.float32),
            pltpu.SemaphoreType.DMA,
            pltpu.SemaphoreType.DMA,
        ],
    )
    def sc_gather(tc_hbm, sc_hbm, x_hbm, bb_hbm, out_hbm,
                  col_v, idx_v, out_v, b_v, sem_tc, sem_sc):
        cls = lax.axis_index("c")
        w2 = lax.axis_index("s")
        tc_cp = pltpu.make_async_copy(
            tc_hbm.at[pl.ds(cls * VT, VT)], col_v.at[pl.ds(0, VT)],
            sem_tc)
        tc_cp.start()
        sc_cp = pltpu.make_async_copy(
            sc_hbm.at[pl.ds(cls * V_SC, V_SC)], col_v.at[pl.ds(VT, V_SC)],
            sem_sc)
        sc_cp.start()
        pltpu.sync_copy(bb_hbm.at[cls], b_v)
        tc_cp.wait()
        sc_cp.wait()
        bvec = b_v[...]
        rowoff = lax.iota(jnp.int32, 16) * L_N

        def grp(g, carry):
            gbase = (w2 * GROUPS_PER_WORKER + g) * GROUP_WORDS
            pltpu.sync_copy(x_hbm.at[pl.ds(gbase, GROUP_WORDS)], idx_v)
            acc = jnp.zeros((LANES,), jnp.float32)
            for j in range(L_N):
                idxs = plsc.load_gather(idx_v, [rowoff + j])
                acc = acc + plsc.load_gather(col_v, [idxs])
            z = acc * jnp.float32(1.0 / L_N) + bvec
            out_v[pl.ds(g * LANES, LANES)] = (
                jnp.float32(1.0) / (jnp.float32(1.0) + jnp.exp(-z)))
            return carry

        lax.fori_loop(0, GROUPS_PER_WORKER, grp, 0)
        pltpu.sync_copy(out_v, out_hbm.at[cls, pl.ds(w2 * ROWS_PER_WORKER,
                                                     ROWS_PER_WORKER)])

    return sc_gather


_sc_gather = _make_sc_gather()


def kernel(x, table, W, b):
    xi = x.astype(jnp.int32).reshape(-1)
    tf = table.astype(jnp.float32)
    Wf = W.astype(jnp.float32)
    Wp = jnp.pad(Wf, ((0, 8 - Wf.shape[0]), (0, 0)))
    small_tc = _project_table_tc(Wp, tf)[:2, :VT].reshape(-1)
    z4 = jnp.zeros((4,), jnp.float32)
    wgflat = jnp.concatenate(
        [Wf[0, :288], z4, Wf[0, 288:], Wf[1, :288], z4, Wf[1, 288:]])
    small_sc = _sc_project(tf, wgflat)
    bb = jnp.broadcast_to(b.astype(jnp.float32)[:, None], (2, LANES))
    out2 = _sc_gather(small_tc, small_sc, xi, bb)
    return out2.T


# speedup vs baseline: 2.2813x; 1.0015x over previous
import functools

import jax
import jax.numpy as jnp
from jax import lax
from jax.experimental import pallas as pl
from jax.experimental.pallas import tpu as pltpu
from jax.experimental.pallas import tpu_sc as plsc

VOCAB_N = 100000
EMB_N = 300
B_N = 4096
L_N = 200

NC = 2
NS = 16
LANES = 16
NW = NC * NS

SCP_PER = 2176
V_SC = NW * SCP_PER
VT = VOCAB_N - V_SC
SCP_GROUPS = SCP_PER // 16
SCP_PAIRS = SCP_GROUPS // 2

CHUNK = 4096
NCHUNK = (VT + CHUNK - 1) // CHUNK
NBUF = 4
_CHUNK_ROWS = [CHUNK] * (NCHUNK - 1) + [VT - CHUNK * (NCHUNK - 1)]
VT_PAD = ((VT + 127) // 128) * 128


def _mm_body(w_ref, t_hbm, o_ref, buf, sems):
    def start(i):
        rows = _CHUNK_ROWS[i]
        pltpu.make_async_copy(
            t_hbm.at[pl.ds(i * CHUNK, rows), :],
            buf.at[i % NBUF, pl.ds(0, rows), :],
            sems.at[i % NBUF]).start()

    for i in range(NBUF):
        start(i)
    for i in range(NCHUNK):
        rows = _CHUNK_ROWS[i]
        pltpu.make_async_copy(
            t_hbm.at[pl.ds(i * CHUNK, rows), :],
            buf.at[i % NBUF, pl.ds(0, rows), :],
            sems.at[i % NBUF]).wait()
        o_ref[:, pl.ds(i * CHUNK, rows)] = lax.dot_general(
            w_ref[...], buf[i % NBUF, pl.ds(0, rows), :],
            dimension_numbers=(((1,), (1,)), ((), ())),
            preferred_element_type=jnp.float32)
        if i + NBUF < NCHUNK:
            start(i + NBUF)
    o_ref[:, 0:1] = jnp.zeros((8, 1), jnp.float32)


def _project_table_tc(Wp, table):
    return pl.pallas_call(
        _mm_body,
        in_specs=[
            pl.BlockSpec(memory_space=pltpu.VMEM),
            pl.BlockSpec(memory_space=pl.ANY),
        ],
        out_specs=pl.BlockSpec(memory_space=pltpu.VMEM),
        out_shape=jax.ShapeDtypeStruct((8, CHUNK * NCHUNK), jnp.float32),
        scratch_shapes=[
            pltpu.VMEM((NBUF, CHUNK, EMB_N), jnp.float32),
            pltpu.SemaphoreType.DMA((NBUF,)),
        ],
    )(Wp, table)


def _make_sc_project():
    mesh = plsc.VectorSubcoreMesh(core_axis_name="c", subcore_axis_name="s")

    @functools.partial(
        pl.kernel,
        mesh=mesh,
        compiler_params=pltpu.CompilerParams(needs_layout_passes=False),
        out_type=jax.ShapeDtypeStruct((2 * V_SC,), jnp.float32),
        scratch_types=[
            pltpu.VMEM((4 * LANES, EMB_N), jnp.float32),
            pltpu.VMEM((4 * LANES, EMB_N), jnp.float32),
            pltpu.VMEM((2 * 304,), jnp.float32),
            pltpu.VMEM((LANES, LANES), jnp.float32),
            pltpu.VMEM((LANES, LANES), jnp.float32),
            pltpu.VMEM((SCP_PER,), jnp.float32),
            pltpu.VMEM((SCP_PER,), jnp.float32),
            pltpu.SemaphoreType.DMA,
            pltpu.SemaphoreType.DMA,
        ],
    )
    def sc_project(t_hbm, wg_hbm, out_hbm, bufa, bufb, wg_v, m0_v, m1_v,
                   c0_v, c1_v, sema, semb):
        wid = lax.axis_index("s") * NC + lax.axis_index("c")
        base = VT + wid * SCP_PER
        pltpu.sync_copy(wg_hbm, wg_v)
        rows16 = lax.iota(jnp.int32, 16)
        NREG = EMB_N // LANES
        w0 = [wg_v[pl.ds(16 * k, 16)] for k in range(NREG + 1)]
        w1 = [wg_v[pl.ds(304 + 16 * k, 16)] for k in range(NREG + 1)]

        BR = 4 * LANES
        NBLK = SCP_PER // BR
        NBPAIR = NBLK // 2

        def start(b, buf, sem):
            pltpu.make_async_copy(
                t_hbm.at[pl.ds(base + b * BR, BR), :], buf, sem).start()

        def wait(b, buf, sem):
            pltpu.make_async_copy(
                t_hbm.at[pl.ds(base + b * BR, BR), :], buf, sem).wait()

        def process(b, buf):
            def one(gg, carry):
                for r in range(LANES):
                    row = gg * LANES + r
                    v0 = buf[row, pl.ds(0, 16)]
                    acc0 = v0 * w0[0]
                    acc1 = v0 * w1[0]
                    for k in range(1, NREG):
                        v = buf[row, pl.ds(16 * k, 16)]
                        acc0 = acc0 + v * w0[k]
                        acc1 = acc1 + v * w1[k]
                    vt = buf[row, pl.ds(EMB_N - 16, 16)]
                    acc0 = acc0 + vt * w0[NREG]
                    acc1 = acc1 + vt * w1[NREG]
                    m0_v[r, pl.ds(0, 16)] = acc0
                    m1_v[r, pl.ds(0, 16)] = acc1
                col0 = jnp.full((LANES,), 0, jnp.int32)
                tot0 = plsc.load_gather(m0_v, [rows16, col0])
                tot1 = plsc.load_gather(m1_v, [rows16, col0])
                for c in range(1, LANES):
                    colc = jnp.full((LANES,), c, jnp.int32)
                    tot0 = tot0 + plsc.load_gather(m0_v, [rows16, colc])
                    tot1 = tot1 + plsc.load_gather(m1_v, [rows16, colc])
                out_off = (b * 4 + gg) * LANES
                c0_v[pl.ds(out_off, LANES)] = tot0
                c1_v[pl.ds(out_off, LANES)] = tot1
                return carry

            lax.fori_loop(0, 4, one, 0)

        start(0, bufa, sema)

        def pair(p, carry):
            b = p * 2
            start(b + 1, bufb, semb)
            wait(b, bufa, sema)
            process(b, bufa)

            @pl.when(p + 1 < NBPAIR)
            def _():
                start(b + 2, bufa, sema)

            wait(b + 1, bufb, semb)
            process(b + 1, bufb)
            return carry

        lax.fori_loop(0, NBPAIR, pair, 0)
        pltpu.sync_copy(c0_v, out_hbm.at[pl.ds(wid * SCP_PER, SCP_PER)])
        pltpu.sync_copy(c1_v, out_hbm.at[pl.ds(V_SC + wid * SCP_PER,
                                               SCP_PER)])

    return sc_project


_sc_project = _make_sc_project()


ROWS_PER_WORKER = B_N // NS
GROUPS_PER_WORKER = ROWS_PER_WORKER // LANES
GROUP_WORDS = LANES * L_N


def _make_sc_gather():
    mesh = plsc.VectorSubcoreMesh(core_axis_name="c", subcore_axis_name="s")

    @functools.partial(
        pl.kernel,
        mesh=mesh,
        compiler_params=pltpu.CompilerParams(needs_layout_passes=False),
        out_type=jax.ShapeDtypeStruct((2, B_N), jnp.float32),
        scratch_types=[
            pltpu.VMEM((VOCAB_N,), jnp.float32),
            pltpu.VMEM((GROUP_WORDS,), jnp.int32),
            pltpu.VMEM((ROWS_PER_WORKER,), jnp.float32),
            pltpu.VMEM((LANES,), jnp.float32),
            pltpu.SemaphoreType.DMA,
            pltpu.SemaphoreType.DMA,
        ],
    )
    def sc_gather(tc_hbm, sc_hbm, x_hbm, bb_hbm, out_hbm,
                  col_v, idx_v, out_v, b_v, sem_tc, sem_sc):
        cls = lax.axis_index("c")
        w2 = lax.axis_index("s")
        tc_cp = pltpu.make_async_copy(
            tc_hbm.at[pl.ds(cls * VT, VT)], col_v.at[pl.ds(0, VT)],
            sem_tc)
        tc_cp.start()
        sc_cp = pltpu.make_async_copy(
            sc_hbm.at[pl.ds(cls * V_SC, V_SC)], col_v.at[pl.ds(VT, V_SC)],
            sem_sc)
        sc_cp.start()
        pltpu.sync_copy(bb_hbm.at[cls], b_v)
        tc_cp.wait()
        sc_cp.wait()
        bvec = b_v[...]
        rowoff = lax.iota(jnp.int32, 16) * L_N

        def grp(g, carry):
            gbase = (w2 * GROUPS_PER_WORKER + g) * GROUP_WORDS
            pltpu.sync_copy(x_hbm.at[pl.ds(gbase, GROUP_WORDS)], idx_v)
            acc = jnp.zeros((LANES,), jnp.float32)
            for j in range(L_N):
                idxs = plsc.load_gather(idx_v, [rowoff + j])
                acc = acc + plsc.load_gather(col_v, [idxs])
            z = acc * jnp.float32(1.0 / L_N) + bvec
            out_v[pl.ds(g * LANES, LANES)] = (
                jnp.float32(1.0) / (jnp.float32(1.0) + jnp.exp(-z)))
            return carry

        lax.fori_loop(0, GROUPS_PER_WORKER, grp, 0)
        pltpu.sync_copy(out_v, out_hbm.at[cls, pl.ds(w2 * ROWS_PER_WORKER,
                                                     ROWS_PER_WORKER)])

    return sc_gather


_sc_gather = _make_sc_gather()


def kernel(x, table, W, b):
    xi = x.astype(jnp.int32).reshape(-1)
    tf = table.astype(jnp.float32)
    Wf = W.astype(jnp.float32)
    Wp = jnp.pad(Wf, ((0, 8 - Wf.shape[0]), (0, 0)))
    small_tc = _project_table_tc(Wp, tf)[:2, :VT].reshape(-1)
    z4 = jnp.zeros((4,), jnp.float32)
    wgflat = jnp.concatenate(
        [Wf[0, :288], z4, Wf[0, 288:], Wf[1, :288], z4, Wf[1, 288:]])
    small_sc = _sc_project(tf, wgflat)
    bb = jnp.broadcast_to(b.astype(jnp.float32)[:, None], (2, LANES))
    out2 = _sc_gather(small_tc, small_sc, xi, bb)
    return out2.T


# speedup vs baseline: 2.7423x; 1.2021x over previous
import functools

import jax
import jax.numpy as jnp
from jax import lax
from jax.experimental import pallas as pl
from jax.experimental.pallas import tpu as pltpu
from jax.experimental.pallas import tpu_sc as plsc

VOCAB_N = 100000
EMB_N = 300
B_N = 4096
L_N = 200

NC = 2
NS = 16
LANES = 16

CHUNK = 5120
NCHUNK = 20
VOCAB_PAD = CHUNK * NCHUNK
NBUF = 4
_CHUNK_ROWS = [CHUNK] * (NCHUNK - 1) + [VOCAB_N - CHUNK * (NCHUNK - 1)]


def _mm_body(w_ref, t_hbm, o_ref, buf, sems):
    def start(i):
        rows = _CHUNK_ROWS[i]
        pltpu.make_async_copy(
            t_hbm.at[pl.ds(i * CHUNK, rows), :],
            buf.at[i % NBUF, pl.ds(0, rows), :],
            sems.at[i % NBUF]).start()

    for i in range(NBUF):
        start(i)
    for i in range(NCHUNK):
        rows = _CHUNK_ROWS[i]
        pltpu.make_async_copy(
            t_hbm.at[pl.ds(i * CHUNK, rows), :],
            buf.at[i % NBUF, pl.ds(0, rows), :],
            sems.at[i % NBUF]).wait()
        o_ref[:, pl.ds(i * CHUNK, CHUNK)] = lax.dot_general(
            w_ref[...], buf[i % NBUF],
            dimension_numbers=(((1,), (1,)), ((), ())),
            preferred_element_type=jnp.float32)
        if i + NBUF < NCHUNK:
            start(i + NBUF)
    o_ref[:, 0:1] = jnp.zeros((8, 1), jnp.float32)


def _project_table(Wp, table):
    return pl.pallas_call(
        _mm_body,
        in_specs=[
            pl.BlockSpec(memory_space=pltpu.VMEM),
            pl.BlockSpec(memory_space=pl.ANY),
        ],
        out_specs=pl.BlockSpec(memory_space=pltpu.VMEM),
        out_shape=jax.ShapeDtypeStruct((8, VOCAB_PAD), jnp.float32),
        scratch_shapes=[
            pltpu.VMEM((NBUF, CHUNK, EMB_N), jnp.float32),
            pltpu.SemaphoreType.DMA((NBUF,)),
        ],
    )(Wp, table)


ROWS_PER_WORKER = B_N // NS
GROUPS_PER_WORKER = ROWS_PER_WORKER // LANES
GROUP_WORDS = LANES * L_N
WORKER_WORDS = ROWS_PER_WORKER * L_N


def _make_sc_kernel():
    mesh = plsc.VectorSubcoreMesh(core_axis_name="c", subcore_axis_name="s")

    @functools.partial(
        pl.kernel,
        mesh=mesh,
        compiler_params=pltpu.CompilerParams(needs_layout_passes=False),
        out_type=jax.ShapeDtypeStruct((2, B_N), jnp.float32),
        scratch_types=[
            pltpu.VMEM((VOCAB_PAD,), jnp.float32),
            pltpu.VMEM((GROUP_WORDS,), jnp.int32),
            pltpu.VMEM((ROWS_PER_WORKER,), jnp.float32),
            pltpu.VMEM((LANES,), jnp.float32),
        ],
    )
    def sc_kernel(small_hbm, x_hbm, bb_hbm, out_hbm, col_v, idx_v, out_v,
                  b_v):
        cls = lax.axis_index("c")
        w2 = lax.axis_index("s")
        pltpu.sync_copy(small_hbm.at[cls], col_v)
        pltpu.sync_copy(bb_hbm.at[cls], b_v)
        bvec = b_v[...]
        rowoff = lax.iota(jnp.int32, 16) * L_N

        def grp(g, carry):
            base = (w2 * GROUPS_PER_WORKER + g) * GROUP_WORDS
            pltpu.sync_copy(x_hbm.at[pl.ds(base, GROUP_WORDS)], idx_v)
            acc = jnp.zeros((LANES,), jnp.float32)
            for j in range(L_N):
                idxs = plsc.load_gather(idx_v, [rowoff + j])
                acc = acc + plsc.load_gather(col_v, [idxs])
            z = acc * jnp.float32(1.0 / L_N) + bvec
            out_v[pl.ds(g * LANES, LANES)] = (
                jnp.float32(1.0) / (jnp.float32(1.0) + jnp.exp(-z)))
            return carry

        lax.fori_loop(0, GROUPS_PER_WORKER, grp, 0)
        pltpu.sync_copy(out_v, out_hbm.at[cls, pl.ds(w2 * ROWS_PER_WORKER,
                                                     ROWS_PER_WORKER)])

    return sc_kernel


_sc_kernel = _make_sc_kernel()


def kernel(x, table, W, b):
    xi = x.astype(jnp.int32).reshape(-1)
    Wp = jnp.pad(W.astype(jnp.float32), ((0, 8 - W.shape[0]), (0, 0)))
    small = _project_table(Wp, table.astype(jnp.float32))
    bb = jnp.broadcast_to(b.astype(jnp.float32)[:, None], (2, LANES))
    out2 = _sc_kernel(small, xi, bb)
    return out2.T


# speedup vs baseline: 2.8262x; 1.0306x over previous
import functools

import jax
import jax.numpy as jnp
from jax import lax
from jax.experimental import pallas as pl
from jax.experimental.pallas import tpu as pltpu
from jax.experimental.pallas import tpu_sc as plsc

VOCAB_N = 100000
EMB_N = 300
B_N = 4096
L_N = 200

NC = 2
NS = 16
LANES = 16

CHUNK = 5120
NCHUNK = 20
VOCAB_PAD = CHUNK * NCHUNK
NBUF = 4
_CHUNK_ROWS = [CHUNK] * (NCHUNK - 1) + [VOCAB_N - CHUNK * (NCHUNK - 1)]


def _mm_body(w_ref, t_hbm, o_ref, buf, sems):
    def start(i):
        rows = _CHUNK_ROWS[i]
        pltpu.make_async_copy(
            t_hbm.at[pl.ds(i * CHUNK, rows), :],
            buf.at[i % NBUF, pl.ds(0, rows), :],
            sems.at[i % NBUF]).start()

    for i in range(NBUF):
        start(i)
    for i in range(NCHUNK):
        rows = _CHUNK_ROWS[i]
        pltpu.make_async_copy(
            t_hbm.at[pl.ds(i * CHUNK, rows), :],
            buf.at[i % NBUF, pl.ds(0, rows), :],
            sems.at[i % NBUF]).wait()
        o_ref[:, pl.ds(i * CHUNK, CHUNK)] = lax.dot_general(
            w_ref[...], buf[i % NBUF],
            dimension_numbers=(((1,), (1,)), ((), ())),
            preferred_element_type=jnp.float32)
        if i + NBUF < NCHUNK:
            start(i + NBUF)
    o_ref[:, 0:1] = jnp.zeros((8, 1), jnp.float32)


def _project_table(Wp, table):
    return pl.pallas_call(
        _mm_body,
        in_specs=[
            pl.BlockSpec(memory_space=pltpu.VMEM),
            pl.BlockSpec(memory_space=pl.ANY),
        ],
        out_specs=pl.BlockSpec(memory_space=pltpu.VMEM),
        out_shape=jax.ShapeDtypeStruct((8, VOCAB_PAD), jnp.float32),
        scratch_shapes=[
            pltpu.VMEM((NBUF, CHUNK, EMB_N), jnp.float32),
            pltpu.SemaphoreType.DMA((NBUF,)),
        ],
    )(Wp, table)


ROWS_PER_WORKER = B_N // NS
GROUPS_PER_WORKER = ROWS_PER_WORKER // LANES
GROUP_WORDS = LANES * L_N
WORKER_WORDS = ROWS_PER_WORKER * L_N


def _make_sc_kernel():
    mesh = plsc.VectorSubcoreMesh(core_axis_name="c", subcore_axis_name="s")

    @functools.partial(
        pl.kernel,
        mesh=mesh,
        compiler_params=pltpu.CompilerParams(needs_layout_passes=False),
        out_type=jax.ShapeDtypeStruct((2, B_N), jnp.float32),
        scratch_types=[
            pltpu.VMEM((VOCAB_PAD,), jnp.float32),
            pltpu.VMEM((GROUP_WORDS,), jnp.int32),
            pltpu.VMEM((GROUP_WORDS,), jnp.int32),
            pltpu.VMEM((ROWS_PER_WORKER,), jnp.float32),
            pltpu.VMEM((LANES,), jnp.float32),
            pltpu.SemaphoreType.DMA,
            pltpu.SemaphoreType.DMA,
        ],
    )
    def sc_kernel(small_hbm, x_hbm, bb_hbm, out_hbm, col_v, idx_a, idx_b,
                  out_v, b_v, sema, semb):
        cls = lax.axis_index("c")
        w2 = lax.axis_index("s")
        pltpu.sync_copy(small_hbm.at[cls], col_v)
        pltpu.sync_copy(bb_hbm.at[cls], b_v)
        bvec = b_v[...]
        rowoff = lax.iota(jnp.int32, 16) * L_N

        def startg(g, buf, sem):
            base = (w2 * GROUPS_PER_WORKER + g) * GROUP_WORDS
            pltpu.make_async_copy(
                x_hbm.at[pl.ds(base, GROUP_WORDS)], buf, sem).start()

        def waitg(g, buf, sem):
            base = (w2 * GROUPS_PER_WORKER + g) * GROUP_WORDS
            pltpu.make_async_copy(
                x_hbm.at[pl.ds(base, GROUP_WORDS)], buf, sem).wait()

        def process(g, buf):
            acc = jnp.zeros((LANES,), jnp.float32)
            for j in range(L_N):
                idxs = plsc.load_gather(buf, [rowoff + j])
                acc = acc + plsc.load_gather(col_v, [idxs])
            z = acc * jnp.float32(1.0 / L_N) + bvec
            out_v[pl.ds(g * LANES, LANES)] = (
                jnp.float32(1.0) / (jnp.float32(1.0) + jnp.exp(-z)))

        startg(0, idx_a, sema)

        def pair(p, carry):
            g = p * 2
            startg(g + 1, idx_b, semb)
            waitg(g, idx_a, sema)
            process(g, idx_a)

            @pl.when(p + 1 < GROUPS_PER_WORKER // 2)
            def _():
                startg(g + 2, idx_a, sema)

            waitg(g + 1, idx_b, semb)
            process(g + 1, idx_b)
            return carry

        lax.fori_loop(0, GROUPS_PER_WORKER // 2, pair, 0)
        pltpu.sync_copy(out_v, out_hbm.at[cls, pl.ds(w2 * ROWS_PER_WORKER,
                                                     ROWS_PER_WORKER)])

    return sc_kernel


_sc_kernel = _make_sc_kernel()


def kernel(x, table, W, b):
    xi = x.astype(jnp.int32).reshape(-1)
    Wp = jnp.pad(W.astype(jnp.float32), ((0, 8 - W.shape[0]), (0, 0)))
    small = _project_table(Wp, table.astype(jnp.float32))
    bb = jnp.broadcast_to(b.astype(jnp.float32)[:, None], (2, LANES))
    out2 = _sc_kernel(small, xi, bb)
    return out2.T
